# baseline (speedup 1.0000x reference)
import functools

import jax
import jax.numpy as jnp
from jax import lax
from jax.experimental import pallas as pl
from jax.experimental.pallas import tpu as pltpu
from jax.experimental.pallas import tpu_sc as plsc

_NC, _NS = 2, 16
_NW = _NC * _NS


def _make_stage_t(V, D):
    DP = 128
    WP = DP + 1
    NWIN = V // DP
    TAIL = V - NWIN * DP
    SR = V * D // DP
    K = (NWIN + _NW - 1) // _NW
    K2 = K + (K % 2)

    mesh = plsc.VectorSubcoreMesh(core_axis_name="c", subcore_axis_name="s")

    @functools.partial(
        pl.kernel,
        mesh=mesh,
        out_type=jax.ShapeDtypeStruct((SR, DP), jnp.float32),
        compiler_params=pltpu.CompilerParams(
            use_tc_tiling_on_sc=True, needs_layout_passes=False),
        scratch_types=[
            pltpu.VMEM((2, D, WP), jnp.float32),
            pltpu.VMEM((2, D, DP), jnp.float32),
            pltpu.SemaphoreType.DMA,
            pltpu.SemaphoreType.DMA,
        ],
    )
    def kt(tt_hbm, tail_hbm, scr_hbm, win_v, tp_v, isem, osem):
        wid = lax.axis_index("s") * _NC + lax.axis_index("c")
        lane = lax.iota(jnp.int32, 16)
        iotas = [lane + 16 * kk for kk in range(D // 16)]

        def w_of(k):
            return k * _NW + wid

        def valid(k):
            return w_of(k) < NWIN

        def in_desc(k, x):
            off = pl.multiple_of(w_of(k) * DP, DP)
            return pltpu.make_async_copy(
                tt_hbm.at[:, pl.ds(off, DP)],
                win_v.at[x, :, pl.ds(0, DP)], isem)

        def out_desc(k, x):
            off = pl.multiple_of(w_of(k) * D, D)
            return pltpu.make_async_copy(
                tp_v.at[x], scr_hbm.at[pl.ds(off, D)], osem)

        def transpose(x):
            @plsc.parallel_loop(0, DP, 1, unroll=8)
            def _(bb):
                col = lane * 0 + bb
                row = lax.shift_right_logical(bb, 1)
                colbase = lax.mul(lax.rem(bb, 2), D)
                for kk in range(D // 16):
                    v = plsc.load_gather(win_v.at[x], [iotas[kk], col])
                    tp_v[x, row, pl.ds(colbase + 16 * kk, 16)] = v

        for b in range(2):
            @pl.when(valid(b))
            def _():
                in_desc(b, b).start()

        def step(k, x):
            @pl.when((k >= 2) & valid(k - 2))
            def _():
                out_desc(k - 2, x).wait()

            @pl.when(valid(k))
            def _():
                in_desc(k, x).wait()
                transpose(x)

                @pl.when(valid(k + 2))
                def _():
                    in_desc(k + 2, x).start()

                out_desc(k, x).start()

        def body(i, carry):
            step(2 * i, 0)
            step(2 * i + 1, 1)
            return carry

        lax.fori_loop(0, K2 // 2, body, 0)
        for k in (K2 - 2, K2 - 1):
            @pl.when(valid(k))
            def _():
                out_desc(k, k % 2).wait()

        @pl.when(wid == 0)
        def _():
            pltpu.sync_copy(tail_hbm, win_v.at[0, :, pl.ds(0, DP)])
            for bb in range(TAIL):
                for kk in range(D // 16):
                    f = bb * D + 16 * kk
                    tp_v[0, f // DP, pl.ds(f % DP, 16)] = (
                        win_v[0, bb, pl.ds(16 * kk, 16)])
            rows = TAIL * D // DP
            pltpu.sync_copy(
                tp_v.at[0, pl.ds(0, rows)],
                scr_hbm.at[pl.ds(SR - rows, rows)])

    return kt


def _make_stage_g(n, V, D, DP):
    per_w = n // _NW
    G = 128
    ng = per_w // G
    NBUF = 8
    LAG = 4

    mesh = plsc.VectorSubcoreMesh(core_axis_name="c", subcore_axis_name="s")

    @functools.partial(
        pl.kernel,
        mesh=mesh,
        out_type=jax.ShapeDtypeStruct((n, DP), jnp.float32),
        compiler_params=pltpu.CompilerParams(use_tc_tiling_on_sc=False),
        scratch_types=[
            pltpu.VMEM((ng, G), jnp.int32),
            pltpu.VMEM((NBUF, G, D), jnp.float32),
            pltpu.SemaphoreType.DMA,
            pltpu.SemaphoreType.DMA,
        ],
    )
    def kg(items_hbm, table_hbm, out_hbm, idx_v, rows_v, gsem, ssem):
        wid = lax.axis_index("s") * _NC + lax.axis_index("c")
        base = pl.multiple_of(wid * per_w, per_w)
        pltpu.sync_copy(items_hbm.at[wid], idx_v)

        def gather_desc(j, slot):
            return pltpu.make_async_copy(
                table_hbm.at[idx_v.at[j]], rows_v.at[slot], gsem)

        def scat_desc(j, slot):
            off = pl.multiple_of(base + j * G, G)
            return pltpu.make_async_copy(
                rows_v.at[slot], out_hbm.at[pl.ds(off, G), pl.ds(0, D)], ssem)

        for b in range(LAG):
            gather_desc(b, b).start()

        def step(j, slot):
            @pl.when(j >= LAG)
            def _():
                scat_desc(j - LAG, (slot + LAG) % NBUF).wait()

            @pl.when(j + LAG < ng)
            def _():
                gather_desc(j + LAG, (slot + LAG) % NBUF).start()

            gather_desc(j, slot).wait()
            scat_desc(j, slot).start()

        def outer(i, carry):
            g = i * NBUF
            for b in range(NBUF):
                step(g + b, b)
            return carry

        lax.fori_loop(0, ng // NBUF, outer, 0)
        for j in range(ng - LAG, ng):
            scat_desc(j, j % NBUF).wait()

    return kg


def kernel(items, table):
    B, T = items.shape
    V, D = table.shape
    DP = 128
    n = B * T
    per_w = n // _NW
    G = 128
    assert n % (_NW * G) == 0 and (per_w // G) % 8 == 0

    NWIN = V // DP
    tail = jnp.pad(table[NWIN * DP:], ((0, 0), (0, DP - D)))
    scratch = _make_stage_t(V, D)(table.T, tail)
    table_c = scratch.reshape(V, D)

    idx = items.reshape(_NW, per_w // G, G).astype(jnp.int32)
    out = _make_stage_g(n, V, D, DP)(idx, table_c)
    return out[:, :D].reshape(B, T, D)


# speedup vs baseline: 1.2144x; 1.2144x over previous
import functools

import jax
import jax.numpy as jnp
from jax import lax
from jax.experimental import pallas as pl
from jax.experimental.pallas import tpu as pltpu
from jax.experimental.pallas import tpu_sc as plsc

_NC, _NS = 2, 16
_NW = _NC * _NS


def _make_sc_gather(n, V, D, DP):
    per_w = n // _NW
    G = 128
    ng = per_w // G
    NBUF = 8
    LAG = 4

    mesh = plsc.VectorSubcoreMesh(core_axis_name="c", subcore_axis_name="s")

    @functools.partial(
        pl.kernel,
        mesh=mesh,
        out_type=jax.ShapeDtypeStruct((n, DP), jnp.float32),
        compiler_params=pltpu.CompilerParams(use_tc_tiling_on_sc=False),
        scratch_types=[
            pltpu.VMEM((ng, G), jnp.int32),
            pltpu.VMEM((NBUF, G, D), jnp.float32),
            pltpu.SemaphoreType.DMA,
            pltpu.SemaphoreType.DMA,
        ],
    )
    def kg(items_hbm, table_hbm, out_hbm, idx_v, rows_v, gsem, ssem):
        wid = lax.axis_index("s") * _NC + lax.axis_index("c")
        base = pl.multiple_of(wid * per_w, per_w)
        tbl = table_hbm
        pltpu.sync_copy(items_hbm.at[wid], idx_v)

        def gather_desc(j, slot):
            return pltpu.make_async_copy(
                tbl.at[idx_v.at[j]], rows_v.at[slot], gsem)

        def scat_desc(j, slot):
            off = pl.multiple_of(base + j * G, G)
            return pltpu.make_async_copy(
                rows_v.at[slot], out_hbm.at[pl.ds(off, G), pl.ds(0, D)], ssem)

        for b in range(LAG):
            gather_desc(b, b).start()

        def step(j, slot):
            @pl.when(j >= LAG)
            def _():
                scat_desc(j - LAG, (slot + LAG) % NBUF).wait()

            @pl.when(j + LAG < ng)
            def _():
                gather_desc(j + LAG, (slot + LAG) % NBUF).start()

            gather_desc(j, slot).wait()
            scat_desc(j, slot).start()

        def outer(i, carry):
            g = i * NBUF
            for b in range(NBUF):
                step(g + b, b)
            return carry

        lax.fori_loop(0, ng // NBUF, outer, 0)
        for j in range(ng - LAG, ng):
            scat_desc(j, j % NBUF).wait()

    return kg


def kernel(items, table):
    B, T = items.shape
    V, D = table.shape
    DP = 128
    n = B * T
    per_w = n // _NW
    G = 128
    assert n % (_NW * G) == 0 and (per_w // G) % 8 == 0 and (V * D) % DP == 0

    table_c = jax.lax.optimization_barrier(table.T).T
    idx = items.reshape(_NW, per_w // G, G).astype(jnp.int32)
    out = _make_sc_gather(n, V, D, DP)(idx, table_c)
    return out[:, :D].reshape(B, T, D)
